# baseline (speedup 1.0000x reference)
import jax
import jax.numpy as jnp
from jax.experimental import pallas as pl
from jax.experimental.pallas import tpu as pltpu

N = 10000
D_IN = 128
D_HID = 64
D_OUT = 32
TOP_K = 32
ROWS = 200
LANES = 128
SLABS = 79
N_PAD = SLABS * LANES
MINI = -2 ** 31


def _mlp_body(x_ref, w1_ref, b1_ref, w2_ref, b2_ref, h_ref):
    x = x_ref[...]
    h1 = jax.lax.dot_general(x, w1_ref[...], (((1,), (0,)), ((), ())),
                             preferred_element_type=jnp.float32)
    h1 = jnp.maximum(h1 + b1_ref[...], 0.0)
    h2 = jax.lax.dot_general(h1, w2_ref[...], (((1,), (0,)), ((), ())),
                             preferred_element_type=jnp.float32)
    h_ref[...] = h2 + b2_ref[...]


def _key_of(f):
    i = jax.lax.bitcast_convert_type(f, jnp.int32)
    return jnp.where(i >= 0, i, i ^ jnp.int32(0x7FFFFFFF))


def _float_of(k):
    i = jnp.where(k >= 0, k, k ^ jnp.int32(0x7FFFFFFF))
    return jax.lax.bitcast_convert_type(i, jnp.float32)


def _topo_body(hb_ref, ha_ref, o_ref, p_scr, cnt_scr, done_scr):
    mini = jnp.int32(MINI)
    hb = hb_ref[...]
    ha = ha_ref[...]
    sim = jax.lax.dot_general(hb, ha, (((1,), (1,)), ((), ())),
                              preferred_element_type=jnp.float32)
    col = jax.lax.broadcasted_iota(jnp.int32, (ROWS, N_PAD), 1)
    sim = jnp.where(col < N, sim, -jnp.inf)

    gm = jnp.max(sim.reshape(ROWS, SLABS, LANES), axis=1)
    ub = jnp.max(gm, axis=1, keepdims=True)
    lb = jnp.min(gm, axis=1, keepdims=True)
    ubk = _key_of(ub)
    lbk = _key_of(lb)

    d = lbk ^ ubk
    df = d.astype(jnp.float32)
    dexp = ((jax.lax.bitcast_convert_type(df, jnp.int32) >> 23) & 255) - 127
    h = jnp.where(d < 0, jnp.int32(31), dexp)
    ubu = ubk ^ mini
    low = (jnp.int32(1) << jnp.clip(h + 1, 0, 31)) - 1
    p0 = jnp.where(h >= 31, jnp.int32(0), ubu & ~low)
    start_bit = jnp.max(jnp.clip(h, 0, 31))

    p_scr[...] = p0
    cnt_scr[...] = jnp.full((ROWS, 1), jnp.int32(0x7FFFFFF), jnp.int32)
    done_scr[0] = jnp.int32(0)
    for bit in range(31, -1, -1):
        v = 1 << bit
        if v >= 2 ** 31:
            v -= 2 ** 32

        @pl.when(jnp.logical_and(bit <= start_bit, done_scr[0] == 0))
        def _(v=v):
            p = p_scr[...]
            c = p | jnp.int32(v)
            cf = _float_of(c ^ mini)
            cnt = jnp.sum((sim >= cf).astype(jnp.int32), axis=1,
                          keepdims=True)
            acc = cnt >= TOP_K
            cnt_new = jnp.where(acc, cnt, cnt_scr[...])
            p_scr[...] = jnp.where(acc, c, p)
            cnt_scr[...] = cnt_new
            done_scr[0] = jnp.all(cnt_new == TOP_K).astype(jnp.int32)

    thresh = _float_of(p_scr[...] ^ mini)
    e = jnp.where(sim >= thresh, jnp.exp(sim - ub), 0.0)
    denom = jnp.sum(e, axis=1, keepdims=True)
    o_ref[...] = (e / denom)[:, :N]


def kernel(node_feat, W1, b1, W2, b2, dense):
    del dense
    b1r = b1.reshape(1, D_HID)
    b2r = b2.reshape(1, D_OUT)
    grid = N // ROWS

    h = pl.pallas_call(
        _mlp_body,
        grid=(grid,),
        in_specs=[
            pl.BlockSpec((ROWS, D_IN), lambda i: (i, 0)),
            pl.BlockSpec((D_IN, D_HID), lambda i: (0, 0)),
            pl.BlockSpec((1, D_HID), lambda i: (0, 0)),
            pl.BlockSpec((D_HID, D_OUT), lambda i: (0, 0)),
            pl.BlockSpec((1, D_OUT), lambda i: (0, 0)),
        ],
        out_specs=pl.BlockSpec((ROWS, D_OUT), lambda i: (i, 0)),
        out_shape=jax.ShapeDtypeStruct((N, D_OUT), jnp.float32),
        compiler_params=pltpu.CompilerParams(
            dimension_semantics=("parallel",)),
    )(node_feat, W1, b1r, W2, b2r)

    h_pad = jnp.pad(h, ((0, N_PAD - N), (0, 0)))

    graph_topo = pl.pallas_call(
        _topo_body,
        grid=(grid,),
        in_specs=[
            pl.BlockSpec((ROWS, D_OUT), lambda i: (i, 0)),
            pl.BlockSpec((N_PAD, D_OUT), lambda i: (0, 0)),
        ],
        out_specs=pl.BlockSpec((ROWS, N), lambda i: (i, 0)),
        out_shape=jax.ShapeDtypeStruct((N, N), jnp.float32),
        scratch_shapes=[
            pltpu.VMEM((ROWS, 1), jnp.int32),
            pltpu.VMEM((ROWS, 1), jnp.int32),
            pltpu.SMEM((1,), jnp.int32),
        ],
        compiler_params=pltpu.CompilerParams(
            dimension_semantics=("parallel",)),
    )(h, h_pad)

    return graph_topo


# speedup vs baseline: 1.1803x; 1.1803x over previous
import jax
import jax.numpy as jnp
from jax.experimental import pallas as pl
from jax.experimental.pallas import tpu as pltpu

N = 10000
D_IN = 128
D_HID = 64
D_OUT = 32
TOP_K = 32
ROWS = 400
LANES = 128
SLABS = 79
N_PAD = SLABS * LANES
MINI = -2 ** 31


def _mlp_body(x_ref, w1_ref, b1_ref, w2_ref, b2_ref, h_ref):
    x = x_ref[...]
    h1 = jax.lax.dot_general(x, w1_ref[...], (((1,), (0,)), ((), ())),
                             preferred_element_type=jnp.float32)
    h1 = jnp.maximum(h1 + b1_ref[...], 0.0)
    h2 = jax.lax.dot_general(h1, w2_ref[...], (((1,), (0,)), ((), ())),
                             preferred_element_type=jnp.float32)
    h_ref[...] = h2 + b2_ref[...]


def _key_of(f):
    i = jax.lax.bitcast_convert_type(f, jnp.int32)
    return jnp.where(i >= 0, i, i ^ jnp.int32(0x7FFFFFFF))


def _float_of(k):
    i = jnp.where(k >= 0, k, k ^ jnp.int32(0x7FFFFFFF))
    return jax.lax.bitcast_convert_type(i, jnp.float32)


def _topo_body(hb_ref, ha_ref, o_ref):
    mini = jnp.int32(MINI)
    hb = hb_ref[...]
    ha = ha_ref[...]
    sim = jax.lax.dot_general(hb, ha, (((1,), (1,)), ((), ())),
                              preferred_element_type=jnp.float32)
    col = jax.lax.broadcasted_iota(jnp.int32, (ROWS, N_PAD), 1)
    sim = jnp.where(col < N, sim, -jnp.inf)

    gm = jnp.max(sim.reshape(ROWS, SLABS, LANES), axis=1)
    ub = jnp.max(gm, axis=1, keepdims=True)
    lb = jnp.min(gm, axis=1, keepdims=True)
    ubk = _key_of(ub)
    lbk = _key_of(lb)

    d = lbk ^ ubk
    df = d.astype(jnp.float32)
    dexp = ((jax.lax.bitcast_convert_type(df, jnp.int32) >> 23) & 255) - 127
    h = jnp.where(d < 0, jnp.int32(31), dexp)
    ubu = ubk ^ mini
    low = (jnp.int32(1) << jnp.clip(h + 1, 0, 31)) - 1
    p0 = jnp.where(h >= 31, jnp.int32(0), ubu & ~low)
    start_bit = jnp.max(jnp.clip(h, 0, 31))

    def cond(state):
        b, _, cntp = state
        return jnp.logical_and(b >= 0, jnp.any(cntp != TOP_K))

    def one_bit(b, p, cntp):
        c = p | (jnp.int32(1) << jnp.maximum(b, 0))
        cf = _float_of(c ^ mini)
        cnt = jnp.sum((sim >= cf).astype(jnp.int32), axis=1, keepdims=True)
        acc = jnp.logical_and(cnt >= TOP_K, b >= 0)
        return jnp.where(acc, c, p), jnp.where(acc, cnt, cntp)

    def body(state):
        b, p, cntp = state
        p, cntp = one_bit(b, p, cntp)
        p, cntp = one_bit(b - 1, p, cntp)
        return (b - 2, p, cntp)

    sentinel = jnp.full((ROWS, 1), jnp.int32(0x7FFFFFF), jnp.int32)
    _, p, _ = jax.lax.while_loop(cond, body, (start_bit, p0, sentinel))

    thresh = _float_of(p ^ mini)
    e = jnp.where(sim >= thresh, jnp.exp(sim - ub), 0.0)
    denom = jnp.sum(e, axis=1, keepdims=True)
    o_ref[...] = (e / denom)[:, :N]


def kernel(node_feat, W1, b1, W2, b2, dense):
    del dense
    b1r = b1.reshape(1, D_HID)
    b2r = b2.reshape(1, D_OUT)
    grid = N // ROWS

    h = pl.pallas_call(
        _mlp_body,
        grid=(grid,),
        in_specs=[
            pl.BlockSpec((ROWS, D_IN), lambda i: (i, 0)),
            pl.BlockSpec((D_IN, D_HID), lambda i: (0, 0)),
            pl.BlockSpec((1, D_HID), lambda i: (0, 0)),
            pl.BlockSpec((D_HID, D_OUT), lambda i: (0, 0)),
            pl.BlockSpec((1, D_OUT), lambda i: (0, 0)),
        ],
        out_specs=pl.BlockSpec((ROWS, D_OUT), lambda i: (i, 0)),
        out_shape=jax.ShapeDtypeStruct((N, D_OUT), jnp.float32),
        compiler_params=pltpu.CompilerParams(
            dimension_semantics=("parallel",)),
    )(node_feat, W1, b1r, W2, b2r)

    h_pad = jnp.pad(h, ((0, N_PAD - N), (0, 0)))

    graph_topo = pl.pallas_call(
        _topo_body,
        grid=(grid,),
        in_specs=[
            pl.BlockSpec((ROWS, D_OUT), lambda i: (i, 0)),
            pl.BlockSpec((N_PAD, D_OUT), lambda i: (0, 0)),
        ],
        out_specs=pl.BlockSpec((ROWS, N), lambda i: (i, 0)),
        out_shape=jax.ShapeDtypeStruct((N, N), jnp.float32),
        compiler_params=pltpu.CompilerParams(
            dimension_semantics=("parallel",)),
    )(h, h_pad)

    return graph_topo
